# baseline (speedup 1.0000x reference)
import functools

import jax
import jax.numpy as jnp
from jax import lax
from jax.experimental import pallas as pl
from jax.experimental.pallas import tpu as pltpu
from jax.experimental.pallas import tpu_sc as plsc

VOCAB = 1000000
DIM = 64
BATCH = 4096
SEQ = 200

_INFO = plsc.get_sparse_core_info()
NC = _INFO.num_cores
NS = _INFO.num_subcores

D_PER_C = DIM // NC
UNITS = -(-SEQ // NS)
NPAIR = (UNITS - 1) // 2

VMAIN = (VOCAB // 128) * 128
VTAIL = VOCAB - VMAIN
STRIP = 62464
REM = VMAIN - NS * STRIP
LINE = VMAIN + 128

_mesh = plsc.VectorSubcoreMesh(core_axis_name="c", subcore_axis_name="s")


@functools.partial(
    pl.kernel,
    mesh=_mesh,
    out_type=jax.ShapeDtypeStruct((SEQ, DIM, BATCH), jnp.float32),
    scratch_types=[
        pltpu.VMEM_SHARED((LINE,), jnp.float32),
        pltpu.VMEM((2 * NPAIR * BATCH,), jnp.int32),
        [pltpu.VMEM((2 * BATCH,), jnp.float32) for _ in range(2)],
        pltpu.SemaphoreType.DMA,
        pltpu.SemaphoreType.DMA,
        [pltpu.SemaphoreType.DMA for _ in range(2)],
        [pltpu.SemaphoreType.DMA for _ in range(2)],
        pltpu.SemaphoreType.DMA,
    ],
    compiler_params=pltpu.CompilerParams(use_tc_tiling_on_sc=True),
)
def _gather_kernel(
    idx3, tab3, tail3, out_t, line, idx_all, gbuf, lsem, isem, gsem, ssem, s12sem
):
    cid = lax.axis_index("c")
    sid = lax.axis_index("s")
    s12 = sid + (UNITS - 1) * NS
    has12 = s12 < SEQ

    @pl.loop(0, 2 * NPAIR)
    def _stage(k):
        s = sid + k * NS
        pltpu.sync_copy(
            idx3.at[s // 8, s % 8, :],
            idx_all.at[pl.ds(pl.multiple_of(k * BATCH, 8), BATCH)],
        )

    d0 = cid * D_PER_C

    def pair_idx(p):
        return idx_all.at[pl.ds(pl.multiple_of(2 * p * BATCH, 8), 2 * BATCH)]

    def slot0_idx():
        return idx_all.at[pl.ds(0, BATCH)]

    def drain(sem_, n):
        for _ in range(n):
            pltpu.make_async_copy(gbuf[0].at[pl.ds(0, BATCH)], out_t.at[0, 0, :], sem_).wait()

    @pl.loop(0, D_PER_C)
    def _d_loop(i):
        d = d0 + i
        dR = d // 8
        dr = d % 8

        off = pl.multiple_of(sid * STRIP, 128)
        pltpu.async_copy(
            tab3.at[dR, dr, pl.ds(off, STRIP)], line.at[pl.ds(off, STRIP)], lsem
        )

        @pl.when(sid == 1)
        def _():
            roff = pl.multiple_of(NS * STRIP, 128)
            pltpu.async_copy(
                tab3.at[dR, dr, pl.ds(roff, REM)], line.at[pl.ds(roff, REM)], lsem
            )
            pltpu.make_async_copy(
                tab3.at[dR, dr, pl.ds(roff, REM)], line.at[pl.ds(roff, REM)], lsem
            ).wait()

        @pl.when(sid == 2)
        def _():
            pltpu.async_copy(tail3.at[dR, dr, :], line.at[pl.ds(VMAIN, 128)], lsem)
            pltpu.make_async_copy(
                tail3.at[dR, dr, :], line.at[pl.ds(VMAIN, 128)], lsem
            ).wait()

        pltpu.make_async_copy(
            tab3.at[dR, dr, pl.ds(off, STRIP)], line.at[pl.ds(off, STRIP)], lsem
        ).wait()
        plsc.subcore_barrier()

        def free_slot(p):
            b = p % 2
            if p >= 2:
                drain(ssem[b], 2)
            elif p == 1:

                @pl.when(i > 0)
                def _():
                    drain(ssem[1], 2)
            else:

                @pl.when(jnp.logical_and(i > 0, has12))
                def _():
                    drain(s12sem, 1)

                @pl.when(jnp.logical_and(i > 0, jnp.logical_not(has12)))
                def _():
                    drain(ssem[0], 2)

        def pair_start(p):
            b = p % 2
            free_slot(p)
            if p == 0:

                @pl.when(jnp.logical_and(i > 0, has12))
                def _():
                    pltpu.make_async_copy(
                        idx3.at[sid // 8, sid % 8, :], slot0_idx(), isem
                    ).wait()

            pltpu.async_copy(line.at[pair_idx(p)], gbuf[b], gsem[b])

        def pair_finish(p):
            b = p % 2
            pltpu.make_async_copy(line.at[pair_idx(p)], gbuf[b], gsem[b]).wait()
            sA = sid + 2 * p * NS
            sB = sA + NS
            pltpu.async_copy(gbuf[b].at[pl.ds(0, BATCH)], out_t.at[sA, d, :], ssem[b])
            pltpu.async_copy(
                gbuf[b].at[pl.ds(BATCH, BATCH)], out_t.at[sB, d, :], ssem[b]
            )
            if p == 0:
                @pl.when(has12)
                def _():
                    pltpu.async_copy(
                        idx3.at[s12 // 8, s12 % 8, :], slot0_idx(), isem
                    )

        pair_start(0)
        for p in range(NPAIR):
            if p + 1 < NPAIR:
                pair_start(p + 1)
            pair_finish(p)

        @pl.when(has12)
        def _():
            drain(ssem[0], 2)
            pltpu.make_async_copy(
                idx3.at[s12 // 8, s12 % 8, :], slot0_idx(), isem
            ).wait()
            pltpu.async_copy(line.at[slot0_idx()], gbuf[0].at[pl.ds(0, BATCH)], gsem[0])
            pltpu.make_async_copy(
                line.at[slot0_idx()], gbuf[0].at[pl.ds(0, BATCH)], gsem[0]
            ).wait()
            pltpu.async_copy(gbuf[0].at[pl.ds(0, BATCH)], out_t.at[s12, d, :], s12sem)
            pltpu.async_copy(idx3.at[sid // 8, sid % 8, :], slot0_idx(), isem)

        plsc.subcore_barrier()

    drain(ssem[1], 2)

    @pl.when(has12)
    def _():
        drain(s12sem, 1)
        pltpu.make_async_copy(
            idx3.at[sid // 8, sid % 8, :], slot0_idx(), isem
        ).wait()

    @pl.when(jnp.logical_not(has12))
    def _():
        drain(ssem[0], 2)


def kernel(input, table):
    tab_t = table.T
    idx3 = input.T.reshape(SEQ // 8, 8, BATCH)
    tab3 = tab_t.reshape(8, DIM // 8, VOCAB)
    tail3 = jnp.pad(tab_t[:, VMAIN:], ((0, 0), (0, 128 - VTAIL))).reshape(
        8, DIM // 8, 128
    )
    out_t = _gather_kernel(idx3, tab3, tail3)
    return out_t.transpose(2, 0, 1)


# speedup vs baseline: 1.0112x; 1.0112x over previous
import functools

import jax
import jax.numpy as jnp
from jax import lax
from jax.experimental import pallas as pl
from jax.experimental.pallas import tpu as pltpu
from jax.experimental.pallas import tpu_sc as plsc

VOCAB = 1000000
DIM = 64
BATCH = 4096
SEQ = 200

_INFO = plsc.get_sparse_core_info()
NC = _INFO.num_cores
NS = _INFO.num_subcores

D_PER_C = DIM // NC
UNITS = -(-SEQ // NS)
NSL = 3
ROUNDS = (UNITS - 1) // NSL

VMAIN = (VOCAB // 128) * 128
VTAIL = VOCAB - VMAIN
STRIP = 62464
REM = VMAIN - NS * STRIP
LINE = VMAIN + 128

_mesh = plsc.VectorSubcoreMesh(core_axis_name="c", subcore_axis_name="s")


@functools.partial(
    pl.kernel,
    mesh=_mesh,
    out_type=jax.ShapeDtypeStruct((SEQ, DIM, BATCH), jnp.float32),
    scratch_types=[
        pltpu.VMEM_SHARED((LINE,), jnp.float32),
        pltpu.VMEM((UNITS * BATCH,), jnp.int32),
        [pltpu.VMEM((BATCH,), jnp.float32) for _ in range(NSL)],
        pltpu.SemaphoreType.DMA,
        [pltpu.SemaphoreType.DMA for _ in range(NSL)],
        [pltpu.SemaphoreType.DMA for _ in range(NSL)],
    ],
    compiler_params=pltpu.CompilerParams(use_tc_tiling_on_sc=True),
)
def _gather_kernel(idx3, tab3, tail3, out_t, line, idx_all, gbuf, lsem, gsem, ssem):
    cid = lax.axis_index("c")
    sid = lax.axis_index("s")

    @pl.loop(0, UNITS)
    def _stage(k):
        s = sid + k * NS

        @pl.when(s < SEQ)
        def _():
            pltpu.sync_copy(
                idx3.at[s // 8, s % 8, :],
                idx_all.at[pl.ds(pl.multiple_of(k * BATCH, 8), BATCH)],
            )

    d0 = cid * D_PER_C

    def idx_of(k):
        return idx_all.at[pl.ds(pl.multiple_of(k * BATCH, 8), BATCH)]

    def drain_store(b):
        pltpu.make_async_copy(gbuf[b], out_t.at[0, 0, :], ssem[b]).wait()

    @pl.loop(0, D_PER_C)
    def _d_loop(i):
        d = d0 + i
        dR = d // 8
        dr = d % 8

        off = pl.multiple_of(sid * STRIP, 128)
        pltpu.async_copy(
            tab3.at[dR, dr, pl.ds(off, STRIP)], line.at[pl.ds(off, STRIP)], lsem
        )

        @pl.when(sid == 1)
        def _():
            roff = pl.multiple_of(NS * STRIP, 128)
            pltpu.async_copy(
                tab3.at[dR, dr, pl.ds(roff, REM)], line.at[pl.ds(roff, REM)], lsem
            )
            pltpu.make_async_copy(
                tab3.at[dR, dr, pl.ds(roff, REM)], line.at[pl.ds(roff, REM)], lsem
            ).wait()

        @pl.when(sid == 2)
        def _():
            pltpu.async_copy(tail3.at[dR, dr, :], line.at[pl.ds(VMAIN, 128)], lsem)
            pltpu.make_async_copy(
                tail3.at[dR, dr, :], line.at[pl.ds(VMAIN, 128)], lsem
            ).wait()

        pltpu.make_async_copy(
            tab3.at[dR, dr, pl.ds(off, STRIP)], line.at[pl.ds(off, STRIP)], lsem
        ).wait()
        plsc.subcore_barrier()

        def unit_start(u, b, first_use):
            s = sid + u * NS

            @pl.when(s < SEQ)
            def _():
                @pl.when(jnp.logical_not(first_use))
                def _():
                    drain_store(b)

                pltpu.async_copy(line.at[idx_of(u)], gbuf[b], gsem[b])

            return s

        def unit_finish(u, b, s):
            @pl.when(s < SEQ)
            def _():
                pltpu.make_async_copy(line.at[idx_of(u)], gbuf[b], gsem[b]).wait()
                pltpu.async_copy(gbuf[b], out_t.at[s, d, :], ssem[b])

        @pl.loop(0, ROUNDS)
        def _rounds(r):
            fresh = jnp.logical_and(i == 0, r == 0)
            ss = [unit_start(r * NSL + b, b, fresh) for b in range(NSL)]
            for b in range(NSL):
                unit_finish(r * NSL + b, b, ss[b])

        s12 = unit_start(UNITS - 1, 0, jnp.bool_(False))
        unit_finish(UNITS - 1, 0, s12)

        plsc.subcore_barrier()

    for b in range(NSL):
        drain_store(b)


def kernel(input, table):
    tab_t = table.T
    idx3 = input.T.reshape(SEQ // 8, 8, BATCH)
    tab3 = tab_t.reshape(8, DIM // 8, VOCAB)
    tail3 = jnp.pad(tab_t[:, VMAIN:], ((0, 0), (0, 128 - VTAIL))).reshape(
        8, DIM // 8, 128
    )
    out_t = _gather_kernel(idx3, tab3, tail3)
    return out_t.transpose(2, 0, 1)
